# baseline (speedup 1.0000x reference)
import functools

import jax
import jax.numpy as jnp
import numpy as np
from jax import lax
from jax.experimental import pallas as pl
from jax.experimental.pallas import tpu as pltpu
from jax.experimental.pallas import tpu_sc as plsc

NUM_WORDS = 1000000
EMBED_DIM = 64
NUM_SAMPLES = 8192
BATCH = 4096
LOG_NUM_WORDS_P1 = float(np.log(NUM_WORDS + 1))

_NC = 2
_NS = 16
_NW = _NC * _NS
_TPW = BATCH // _NW
_SPW = NUM_SAMPLES // _NW
_IDS_PER_W = _TPW + _SPW
_CHUNK = 128
_NCHUNK = _IDS_PER_W // _CHUNK
_GC = 16
_NGC = _IDS_PER_W // _GC


def _sc_gather_body(w_hbm, b_hbm, t_hbm, s_hbm,
                    out_tw, out_sw, out_tb, out_sb,
                    ids_v, tiles_v, rows_v, bias_v, sem_w0, sem_w1, sem_b):
    wid = lax.axis_index("s") * _NC + lax.axis_index("c")
    pltpu.sync_copy(t_hbm.at[pl.ds(wid * _TPW, _TPW)],
                    ids_v.at[pl.ds(0, _TPW)])
    pltpu.sync_copy(s_hbm.at[pl.ds(wid * _SPW, _SPW)],
                    ids_v.at[pl.ds(_TPW, _SPW)])
    bcps = [pltpu.async_copy(b_hbm.at[ids_v.at[pl.ds(j * _CHUNK, _CHUNK)]],
                             bias_v.at[pl.ds(j * _CHUNK, _CHUNK)], sem_b)
            for j in range(_NCHUNK)]

    def issue(c, slot, sem):
        idvec = ids_v[pl.ds(c * _GC, _GC)]
        tvec = lax.shift_right_logical(idvec, 3)
        for k in range(_GC):
            pltpu.async_copy(w_hbm.at[pl.ds(tvec[k] * 8, 8), :],
                             tiles_v.at[slot, pl.ds(k * 8, 8)], sem)

    def drain_select(c, slot, sem):
        pltpu.make_async_copy(w_hbm.at[pl.ds(0, _GC * 8), :],
                              tiles_v.at[slot], sem).wait()
        pvec = ids_v[pl.ds(c * _GC, _GC)] & 7
        for k in range(_GC):
            p = pvec[k]
            for g in range(EMBED_DIM // 16):
                rows_v[c * _GC + k, pl.ds(g * 16, 16)] = (
                    tiles_v[slot, k * 8 + p, pl.ds(g * 16, 16)])

    issue(0, 0, sem_w0)

    def body(i, carry):
        c0 = i * 2

        @pl.when(c0 + 1 < _NGC)
        def _():
            issue(c0 + 1, 1, sem_w1)

        drain_select(c0, 0, sem_w0)

        @pl.when(c0 + 2 < _NGC)
        def _():
            issue(c0 + 2, 0, sem_w0)

        drain_select(c0 + 1, 1, sem_w1)
        return carry

    lax.fori_loop(0, _NGC // 2, body, 0)
    for cp in bcps:
        cp.wait()
    pltpu.sync_copy(rows_v.at[pl.ds(0, _TPW)],
                    out_tw.at[pl.ds(wid * _TPW, _TPW)])
    pltpu.sync_copy(rows_v.at[pl.ds(_TPW, _SPW)],
                    out_sw.at[pl.ds(wid * _SPW, _SPW)])
    pltpu.sync_copy(bias_v.at[pl.ds(0, _TPW)],
                    out_tb.at[pl.ds(wid * _TPW, _TPW)])
    pltpu.sync_copy(bias_v.at[pl.ds(_TPW, _SPW)],
                    out_sb.at[pl.ds(wid * _SPW, _SPW)])


def _sc_gather(softmax_w3, softmax_b, targets, sampled_ids):
    mesh = plsc.VectorSubcoreMesh(core_axis_name="c", subcore_axis_name="s")
    return pl.kernel(
        _sc_gather_body,
        out_type=(
            jax.ShapeDtypeStruct((BATCH, EMBED_DIM), jnp.float32),
            jax.ShapeDtypeStruct((NUM_SAMPLES, EMBED_DIM), jnp.float32),
            jax.ShapeDtypeStruct((BATCH,), jnp.float32),
            jax.ShapeDtypeStruct((NUM_SAMPLES,), jnp.float32),
        ),
        mesh=mesh,
        scratch_types=[
            pltpu.VMEM((_IDS_PER_W,), jnp.int32),
            pltpu.VMEM((2, _GC * 8, EMBED_DIM), jnp.float32),
            pltpu.VMEM((_IDS_PER_W, EMBED_DIM), jnp.float32),
            pltpu.VMEM((_IDS_PER_W,), jnp.float32),
            pltpu.SemaphoreType.DMA,
            pltpu.SemaphoreType.DMA,
            pltpu.SemaphoreType.DMA,
        ],
    )(softmax_w3, softmax_b, targets, sampled_ids)


_TB = 4096


def _tr_body(wt_ref, out_ref):
    out_ref[...] = wt_ref[...].T


def _tc_transpose(wt):
    n = wt.shape[1]
    return pl.pallas_call(
        _tr_body,
        grid=(pl.cdiv(n, _TB),),
        in_specs=[pl.BlockSpec((EMBED_DIM, _TB), lambda i: (0, i))],
        out_specs=pl.BlockSpec((_TB, EMBED_DIM), lambda i: (i, 0)),
        out_shape=jax.ShapeDtypeStruct((n, EMBED_DIM), jnp.float32),
    )(wt)


_BB = 256
_NB = BATCH // _BB


def _tc_loss_body(nt_ref, emb_ref, tw_ref, tb_ref, t_ref,
                  sw_ref, sb_ref, sid_ref, out_ref):
    i = pl.program_id(0)
    nt = nt_ref[0, 0]
    e = emb_ref[...]
    tw = tw_ref[...]
    tb = tb_ref[...]
    t = t_ref[...]
    sw = sw_ref[...]
    sb = sb_ref[...]
    sid = sid_ref[...]

    tf = t.astype(jnp.float32)
    tp = jnp.log((tf + 2.0) / (tf + 1.0)) / LOG_NUM_WORDS_P1
    tec = -1.0 * (jnp.exp(nt * jnp.log1p(-tp)) - 1.0)
    true_logit = (jnp.sum(tw * e, axis=1, keepdims=True) + tb
                  - jnp.log(tec + 1e-07))

    sf = sid.astype(jnp.float32)
    sp = jnp.log((sf + 2.0) / (sf + 1.0)) / LOG_NUM_WORDS_P1
    sec = -1.0 * (jnp.exp(nt * jnp.log1p(-sp)) - 1.0)
    sadj = sb - jnp.log(sec + 1e-07)

    logits = lax.dot_general(e, sw, (((1,), (1,)), ((), ())),
                             preferred_element_type=jnp.float32)
    logits = logits + sadj
    logits = jnp.where(sid == t, -10000.0, logits)

    m = jnp.maximum(jnp.max(logits, axis=1, keepdims=True), true_logit)
    s = (jnp.sum(jnp.exp(logits - m), axis=1, keepdims=True)
         + jnp.exp(true_logit - m))
    lse = m + jnp.log(s)
    part = jnp.sum(lse - true_logit)

    @pl.when(i == 0)
    def _():
        out_ref[0, 0] = part

    @pl.when(i != 0)
    def _():
        out_ref[0, 0] = out_ref[0, 0] + part


def _tc_loss(nt, emb, tw, tb, t2, sw, sb2, sid2, interpret=False):
    return pl.pallas_call(
        _tc_loss_body,
        grid=(_NB,),
        in_specs=[
            pl.BlockSpec(memory_space=pltpu.SMEM),
            pl.BlockSpec((_BB, EMBED_DIM), lambda i: (i, 0)),
            pl.BlockSpec((_BB, EMBED_DIM), lambda i: (i, 0)),
            pl.BlockSpec((_BB, 1), lambda i: (i, 0)),
            pl.BlockSpec((_BB, 1), lambda i: (i, 0)),
            pl.BlockSpec((NUM_SAMPLES, EMBED_DIM), lambda i: (0, 0)),
            pl.BlockSpec((1, NUM_SAMPLES), lambda i: (0, 0)),
            pl.BlockSpec((1, NUM_SAMPLES), lambda i: (0, 0)),
        ],
        out_specs=pl.BlockSpec(memory_space=pltpu.SMEM),
        out_shape=jax.ShapeDtypeStruct((1, 1), jnp.float32),
        interpret=interpret,
    )(nt, emb, tw, tb, t2, sw, sb2, sid2)


def kernel(embeddings, targets, softmax_w, softmax_b, sampled_ids, num_tries):
    w_rm = _tc_transpose(softmax_w.T)
    tw, sw, tb1, sb1 = _sc_gather(w_rm, softmax_b,
                                  targets, sampled_ids)
    tb = tb1.reshape(BATCH, 1)
    sb2 = sb1.reshape(1, NUM_SAMPLES)
    t2 = targets.reshape(BATCH, 1)
    sid2 = sampled_ids.reshape(1, NUM_SAMPLES)
    nt = jnp.asarray(num_tries, jnp.float32).reshape(1, 1)
    loss = _tc_loss(nt, embeddings, tw, tb, t2, sw, sb2, sid2)
    return loss[0, 0]


# speedup vs baseline: 1.0003x; 1.0003x over previous
import functools

import jax
import jax.numpy as jnp
import numpy as np
from jax import lax
from jax.experimental import pallas as pl
from jax.experimental.pallas import tpu as pltpu
from jax.experimental.pallas import tpu_sc as plsc

NUM_WORDS = 1000000
EMBED_DIM = 64
NUM_SAMPLES = 8192
BATCH = 4096
LOG_NUM_WORDS_P1 = float(np.log(NUM_WORDS + 1))

_NC = 2
_NS = 16
_NW = _NC * _NS
_TPW = BATCH // _NW
_SPW = NUM_SAMPLES // _NW
_IDS_PER_W = _TPW + _SPW
_CHUNK = 128
_NCHUNK = _IDS_PER_W // _CHUNK
_GC = 16
_NGC = _IDS_PER_W // _GC


def _sc_gather_body(w_hbm, b_hbm, t_hbm, s_hbm,
                    out_tw, out_sw, out_tb, out_sb,
                    ids_v, tiles_v, rows_v, bias_v, sem_w0, sem_w1, sem_b):
    wid = lax.axis_index("s") * _NC + lax.axis_index("c")
    pltpu.sync_copy(t_hbm.at[pl.ds(wid * _TPW, _TPW)],
                    ids_v.at[pl.ds(0, _TPW)])
    pltpu.sync_copy(s_hbm.at[pl.ds(wid * _SPW, _SPW)],
                    ids_v.at[pl.ds(_TPW, _SPW)])
    bcps = [pltpu.async_copy(b_hbm.at[ids_v.at[pl.ds(j * _CHUNK, _CHUNK)]],
                             bias_v.at[pl.ds(j * _CHUNK, _CHUNK)], sem_b)
            for j in range(_NCHUNK)]

    def issue(c, slot, sem):
        idvec = ids_v[pl.ds(c * _GC, _GC)]
        tvec = lax.shift_right_logical(idvec, 3)
        for k in range(_GC):
            pltpu.async_copy(w_hbm.at[pl.ds(tvec[k] * 8, 8), :],
                             tiles_v.at[slot, pl.ds(k * 8, 8)], sem)

    def drain_select(c, slot, sem):
        pltpu.make_async_copy(w_hbm.at[pl.ds(0, _GC * 8), :],
                              tiles_v.at[slot], sem).wait()
        pvec = ids_v[pl.ds(c * _GC, _GC)] & 7
        for k in range(_GC):
            p = pvec[k]
            for g in range(EMBED_DIM // 16):
                rows_v[c * _GC + k, pl.ds(g * 16, 16)] = (
                    tiles_v[slot, k * 8 + p, pl.ds(g * 16, 16)])

    issue(0, 0, sem_w0)

    def body(i, carry):
        c0 = i * 2

        @pl.when(c0 + 1 < _NGC)
        def _():
            issue(c0 + 1, 1, sem_w1)

        drain_select(c0, 0, sem_w0)

        @pl.when(c0 + 2 < _NGC)
        def _():
            issue(c0 + 2, 0, sem_w0)

        drain_select(c0 + 1, 1, sem_w1)
        return carry

    lax.fori_loop(0, _NGC // 2, body, 0)
    for cp in bcps:
        cp.wait()
    pltpu.sync_copy(rows_v.at[pl.ds(0, _TPW)],
                    out_tw.at[pl.ds(wid * _TPW, _TPW)])
    pltpu.sync_copy(rows_v.at[pl.ds(_TPW, _SPW)],
                    out_sw.at[pl.ds(wid * _SPW, _SPW)])
    pltpu.sync_copy(bias_v.at[pl.ds(0, _TPW)],
                    out_tb.at[pl.ds(wid * _TPW, _TPW)])
    pltpu.sync_copy(bias_v.at[pl.ds(_TPW, _SPW)],
                    out_sb.at[pl.ds(wid * _SPW, _SPW)])


def _sc_gather(softmax_w3, softmax_b, targets, sampled_ids):
    mesh = plsc.VectorSubcoreMesh(core_axis_name="c", subcore_axis_name="s")
    return pl.kernel(
        _sc_gather_body,
        out_type=(
            jax.ShapeDtypeStruct((BATCH, EMBED_DIM), jnp.float32),
            jax.ShapeDtypeStruct((NUM_SAMPLES, EMBED_DIM), jnp.float32),
            jax.ShapeDtypeStruct((BATCH,), jnp.float32),
            jax.ShapeDtypeStruct((NUM_SAMPLES,), jnp.float32),
        ),
        mesh=mesh,
        scratch_types=[
            pltpu.VMEM((_IDS_PER_W,), jnp.int32),
            pltpu.VMEM((2, _GC * 8, EMBED_DIM), jnp.float32),
            pltpu.VMEM((_IDS_PER_W, EMBED_DIM), jnp.float32),
            pltpu.VMEM((_IDS_PER_W,), jnp.float32),
            pltpu.SemaphoreType.DMA,
            pltpu.SemaphoreType.DMA,
            pltpu.SemaphoreType.DMA,
        ],
        compiler_params=pltpu.CompilerParams(needs_layout_passes=False),
    )(softmax_w3, softmax_b, targets, sampled_ids)


_TB = 4096


def _tr_body(wt_ref, out_ref):
    out_ref[...] = wt_ref[...].T


def _tc_transpose(wt):
    n = wt.shape[1]
    return pl.pallas_call(
        _tr_body,
        grid=(pl.cdiv(n, _TB),),
        in_specs=[pl.BlockSpec((EMBED_DIM, _TB), lambda i: (0, i))],
        out_specs=pl.BlockSpec((_TB, EMBED_DIM), lambda i: (i, 0)),
        out_shape=jax.ShapeDtypeStruct((n, EMBED_DIM), jnp.float32),
    )(wt)


_BB = 256
_NB = BATCH // _BB


def _tc_loss_body(nt_ref, emb_ref, tw_ref, tb_ref, t_ref,
                  sw_ref, sb_ref, sid_ref, out_ref):
    i = pl.program_id(0)
    nt = nt_ref[0, 0]
    e = emb_ref[...]
    tw = tw_ref[...]
    tb = tb_ref[...]
    t = t_ref[...]
    sw = sw_ref[...]
    sb = sb_ref[...]
    sid = sid_ref[...]

    tf = t.astype(jnp.float32)
    tp = jnp.log((tf + 2.0) / (tf + 1.0)) / LOG_NUM_WORDS_P1
    tec = -1.0 * (jnp.exp(nt * jnp.log1p(-tp)) - 1.0)
    true_logit = (jnp.sum(tw * e, axis=1, keepdims=True) + tb
                  - jnp.log(tec + 1e-07))

    sf = sid.astype(jnp.float32)
    sp = jnp.log((sf + 2.0) / (sf + 1.0)) / LOG_NUM_WORDS_P1
    sec = -1.0 * (jnp.exp(nt * jnp.log1p(-sp)) - 1.0)
    sadj = sb - jnp.log(sec + 1e-07)

    logits = lax.dot_general(e, sw, (((1,), (1,)), ((), ())),
                             preferred_element_type=jnp.float32)
    logits = logits + sadj
    logits = jnp.where(sid == t, -10000.0, logits)

    m = jnp.maximum(jnp.max(logits, axis=1, keepdims=True), true_logit)
    s = (jnp.sum(jnp.exp(logits - m), axis=1, keepdims=True)
         + jnp.exp(true_logit - m))
    lse = m + jnp.log(s)
    part = jnp.sum(lse - true_logit)

    @pl.when(i == 0)
    def _():
        out_ref[0, 0] = part

    @pl.when(i != 0)
    def _():
        out_ref[0, 0] = out_ref[0, 0] + part


def _tc_loss(nt, emb, tw, tb, t2, sw, sb2, sid2, interpret=False):
    return pl.pallas_call(
        _tc_loss_body,
        grid=(_NB,),
        in_specs=[
            pl.BlockSpec(memory_space=pltpu.SMEM),
            pl.BlockSpec((_BB, EMBED_DIM), lambda i: (i, 0)),
            pl.BlockSpec((_BB, EMBED_DIM), lambda i: (i, 0)),
            pl.BlockSpec((_BB, 1), lambda i: (i, 0)),
            pl.BlockSpec((_BB, 1), lambda i: (i, 0)),
            pl.BlockSpec((NUM_SAMPLES, EMBED_DIM), lambda i: (0, 0)),
            pl.BlockSpec((1, NUM_SAMPLES), lambda i: (0, 0)),
            pl.BlockSpec((1, NUM_SAMPLES), lambda i: (0, 0)),
        ],
        out_specs=pl.BlockSpec(memory_space=pltpu.SMEM),
        out_shape=jax.ShapeDtypeStruct((1, 1), jnp.float32),
        interpret=interpret,
    )(nt, emb, tw, tb, t2, sw, sb2, sid2)


def kernel(embeddings, targets, softmax_w, softmax_b, sampled_ids, num_tries):
    tw, sw, tb1, sb1 = _sc_gather(softmax_w, softmax_b,
                                  targets, sampled_ids)
    tb = tb1.reshape(BATCH, 1)
    sb2 = sb1.reshape(1, NUM_SAMPLES)
    t2 = targets.reshape(BATCH, 1)
    sid2 = sampled_ids.reshape(1, NUM_SAMPLES)
    nt = jnp.asarray(num_tries, jnp.float32).reshape(1, 1)
    loss = _tc_loss(nt, embeddings, tw, tb, t2, sw, sb2, sid2)
    return loss[0, 0]


# speedup vs baseline: 1.2966x; 1.2963x over previous
import jax
import jax.numpy as jnp
import numpy as np
from jax import lax
from jax.experimental import pallas as pl
from jax.experimental.pallas import tpu as pltpu
from jax.experimental.pallas import tpu_sc as plsc

NUM_WORDS = 1000000
EMBED_DIM = 64
NUM_SAMPLES = 8192
BATCH = 4096
LOG_NUM_WORDS_P1 = float(np.log(NUM_WORDS + 1))

_NC = 2
_NS = 16
_NW = _NC * _NS
_TOTAL_IDS = BATCH + NUM_SAMPLES
_IDS_PER_W = _TOTAL_IDS // _NW
_CHUNK = 128
_NCHUNK = _IDS_PER_W // _CHUNK

_GC = 16
_NGC = _IDS_PER_W // _GC


def _sc_gather_body(w_hbm, b_hbm, ids_hbm, out_w, out_b,
                    ids_v, tiles_v, rows_v, bias_v, sem_w, sem_b):
    wid = lax.axis_index("s") * _NC + lax.axis_index("c")
    base = wid * _IDS_PER_W
    pltpu.sync_copy(ids_hbm.at[pl.ds(base, _IDS_PER_W)], ids_v)
    bcps = [pltpu.async_copy(b_hbm.at[ids_v.at[pl.ds(j * _CHUNK, _CHUNK)]],
                             bias_v.at[pl.ds(j * _CHUNK, _CHUNK)], sem_b)
            for j in range(_NCHUNK)]

    def body(c, carry):
        idvec = ids_v[pl.ds(c * _GC, _GC)]
        tvec = jax.lax.shift_right_logical(idvec, 3)
        pvec = idvec & 7
        cps = []
        for k in range(_GC):
            cps.append(pltpu.async_copy(
                w_hbm.at[tvec[k]], tiles_v.at[k], sem_w))
        for cp in cps:
            cp.wait()
        for k in range(_GC):
            p = pvec[k]
            for g in range(EMBED_DIM // 16):
                rows_v[c * _GC + k, pl.ds(g * 16, 16)] = (
                    tiles_v[k, p, pl.ds(g * 16, 16)])
        return carry

    jax.lax.fori_loop(0, _NGC, body, 0)
    for cp in bcps:
        cp.wait()
    pltpu.sync_copy(rows_v, out_w.at[pl.ds(base, _IDS_PER_W)])
    pltpu.sync_copy(bias_v, out_b.at[pl.ds(base, _IDS_PER_W)])


def _sc_gather(softmax_w3, softmax_b, all_ids):
    mesh = plsc.VectorSubcoreMesh(core_axis_name="c", subcore_axis_name="s")
    return pl.kernel(
        _sc_gather_body,
        out_type=(
            jax.ShapeDtypeStruct((_TOTAL_IDS, EMBED_DIM), jnp.float32),
            jax.ShapeDtypeStruct((_TOTAL_IDS,), jnp.float32),
        ),
        mesh=mesh,
        scratch_types=[
            pltpu.VMEM((_IDS_PER_W,), jnp.int32),
            pltpu.VMEM((_GC, 8, EMBED_DIM), jnp.float32),
            pltpu.VMEM((_IDS_PER_W, EMBED_DIM), jnp.float32),
            pltpu.VMEM((_IDS_PER_W,), jnp.float32),
            pltpu.SemaphoreType.DMA,
            pltpu.SemaphoreType.DMA,
        ],
    )(softmax_w3, softmax_b, all_ids)


_BB = 256
_NB = BATCH // _BB


def _tc_loss_body(nt_ref, emb_ref, tw_ref, tb_ref, t_ref,
                  sw_ref, sb_ref, sid_ref, out_ref):
    i = pl.program_id(0)
    nt = nt_ref[0, 0]
    e = emb_ref[...]
    tw = tw_ref[...]
    tb = tb_ref[...]
    t = t_ref[...]
    sw = sw_ref[...]
    sb = sb_ref[...]
    sid = sid_ref[...]

    tf = t.astype(jnp.float32)
    tp = jnp.log((tf + 2.0) / (tf + 1.0)) / LOG_NUM_WORDS_P1
    tec = -1.0 * (jnp.exp(nt * jnp.log1p(-tp)) - 1.0)
    true_logit = (jnp.sum(tw * e, axis=1, keepdims=True) + tb
                  - jnp.log(tec + 1e-07))

    sf = sid.astype(jnp.float32)
    sp = jnp.log((sf + 2.0) / (sf + 1.0)) / LOG_NUM_WORDS_P1
    sec = -1.0 * (jnp.exp(nt * jnp.log1p(-sp)) - 1.0)
    sadj = sb - jnp.log(sec + 1e-07)

    logits = lax.dot_general(e, sw, (((1,), (1,)), ((), ())),
                             preferred_element_type=jnp.float32)
    logits = logits + sadj
    logits = jnp.where(sid == t, -10000.0, logits)

    m = jnp.maximum(jnp.max(logits, axis=1, keepdims=True), true_logit)
    s = (jnp.sum(jnp.exp(logits - m), axis=1, keepdims=True)
         + jnp.exp(true_logit - m))
    lse = m + jnp.log(s)
    part = jnp.sum(lse - true_logit)

    @pl.when(i == 0)
    def _():
        out_ref[0, 0] = part

    @pl.when(i != 0)
    def _():
        out_ref[0, 0] = out_ref[0, 0] + part


def _tc_loss(nt, emb, tw, tb, t2, sw, sb2, sid2, interpret=False):
    return pl.pallas_call(
        _tc_loss_body,
        grid=(_NB,),
        in_specs=[
            pl.BlockSpec(memory_space=pltpu.SMEM),
            pl.BlockSpec((_BB, EMBED_DIM), lambda i: (i, 0)),
            pl.BlockSpec((_BB, EMBED_DIM), lambda i: (i, 0)),
            pl.BlockSpec((_BB, 1), lambda i: (i, 0)),
            pl.BlockSpec((_BB, 1), lambda i: (i, 0)),
            pl.BlockSpec((NUM_SAMPLES, EMBED_DIM), lambda i: (0, 0)),
            pl.BlockSpec((1, NUM_SAMPLES), lambda i: (0, 0)),
            pl.BlockSpec((1, NUM_SAMPLES), lambda i: (0, 0)),
        ],
        out_specs=pl.BlockSpec(memory_space=pltpu.SMEM),
        out_shape=jax.ShapeDtypeStruct((1, 1), jnp.float32),
        interpret=interpret,
    )(nt, emb, tw, tb, t2, sw, sb2, sid2)


def kernel(embeddings, targets, softmax_w, softmax_b, sampled_ids, num_tries):
    all_ids = jnp.concatenate([targets, sampled_ids], axis=0)
    all_w, all_b = _sc_gather(softmax_w.reshape(NUM_WORDS // 8, 8, EMBED_DIM),
                              softmax_b, all_ids)
    tw = all_w[:BATCH]
    sw = all_w[BATCH:]
    tb = all_b[:BATCH].reshape(BATCH, 1)
    sb2 = all_b[BATCH:].reshape(1, NUM_SAMPLES)
    t2 = targets.reshape(BATCH, 1)
    sid2 = sampled_ids.reshape(1, NUM_SAMPLES)
    nt = jnp.asarray(num_tries, jnp.float32).reshape(1, 1)
    loss = _tc_loss(nt, embeddings, tw, tb, t2, sw, sb2, sid2)
    return loss[0, 0]
